# Initial kernel scaffold; baseline (speedup 1.0000x reference)
import jax
import jax.numpy as jnp
from jax.experimental import pallas as pl


def kernel(byte_repr, patch_boundaries, Wq, Wk, Wv, bq, bk, bv, Wo, bo):
    raise NotImplementedError("write your pallas kernel here")


# speedup vs baseline: 5.6246x; 5.6246x over previous
import jax
import jax.numpy as jnp
from jax.experimental import pallas as pl

B, L, D, H = 8, 2048, 768, 8
DH = D // H
ST = 256
NT = L // ST
SCALE = 1.0 / (DH ** 0.5)
F32 = jnp.float32
BF16 = jnp.bfloat16


def _dot(a, b):
    return jax.lax.dot_general(a, b, (((1,), (0,)), ((), ())),
                               preferred_element_type=F32)


def _dot_t(a, b):
    return jax.lax.dot_general(a, b, (((0,), (0,)), ((), ())),
                               preferred_element_type=F32)


def _body(x_ref, pb_ref, wq_ref, wk_ref, wv_ref, wo_ref,
          bq_ref, bk_ref, bv_ref, bo_ref, o_ref):
    x = x_ref[0]
    pb = pb_ref[0]

    bnd = (pb != 0).astype(BF16)
    seg_parts = []
    for t in range(NT):
        tril = (jax.lax.broadcasted_iota(jnp.int32, (L, ST), 0) <=
                jax.lax.broadcasted_iota(jnp.int32, (L, ST), 1) + t * ST
                ).astype(BF16)
        seg_parts.append(_dot(bnd, tril))
    seg = jnp.concatenate(seg_parts, axis=1) - bnd[:, :1].astype(F32)
    seg_i = seg.astype(jnp.int32)

    def onehot(t):
        i0 = jax.lax.broadcasted_iota(jnp.int32, (ST, L), 0) + t * ST
        return (i0 == jnp.broadcast_to(seg_i, (ST, L))).astype(BF16)

    ones_col = jnp.ones((L, 1), BF16)

    xq = _dot(x, wq_ref[...]).astype(BF16)
    q_tiles, cnt_tiles = [], []
    for t in range(NT):
        p = onehot(t)
        cnt = _dot(p, ones_col)
        sums = _dot(p, xq)
        q = sums / jnp.maximum(cnt, 1.0) + bq_ref[...]
        q_tiles.append(q.astype(BF16))
        cnt_tiles.append(cnt)

    qj = sum(_dot_t(onehot(t), q_tiles[t]) for t in range(NT))

    e_mat = (jax.lax.broadcasted_iota(jnp.int32, (D, H), 0) // DH ==
             jax.lax.broadcasted_iota(jnp.int32, (D, H), 1)).astype(F32)
    logit_tiles = []
    for t in range(NT):
        sl = slice(t * ST, (t + 1) * ST)
        k_t = _dot(x[sl], wk_ref[...]) + bk_ref[...]
        logit_tiles.append(_dot(qj[sl] * k_t, e_mat) * SCALE)
    logits = jnp.concatenate(logit_tiles, axis=0)
    m = jnp.max(logits, axis=0, keepdims=True)
    e = jnp.exp(logits - m)

    escale = jnp.zeros((L, H), F32)
    for t in range(NT):
        p = onehot(t)
        dinv = 1.0 / jnp.maximum(_dot(p, e.astype(BF16)), 1e-30)
        escale = escale + _dot_t(p, dinv.astype(BF16))

    w = _dot((e * escale).astype(BF16), e_mat.T.astype(BF16))
    yv_tiles = []
    for t in range(NT):
        sl = slice(t * ST, (t + 1) * ST)
        v_t = _dot(x[sl], wv_ref[...]) + bv_ref[...]
        yv_tiles.append((w[sl] * v_t).astype(BF16))
    yv = jnp.concatenate(yv_tiles, axis=0)

    for t in range(NT):
        attn = _dot(onehot(t), yv).astype(BF16)
        out = _dot(attn, wo_ref[...]) + bo_ref[...]
        out = jnp.where(cnt_tiles[t] > 0.0, out, 0.0)
        o_ref[0, t * ST:(t + 1) * ST, :] = out


@jax.jit
def kernel(byte_repr, patch_boundaries, Wq, Wk, Wv, bq, bk, bv, Wo, bo):
    pb3 = patch_boundaries.reshape(B, 1, L)
    full = lambda shape: pl.BlockSpec(shape, lambda b: (0,) * len(shape))
    out = pl.pallas_call(
        _body,
        grid=(B,),
        in_specs=[
            pl.BlockSpec((1, L, D), lambda b: (b, 0, 0)),
            pl.BlockSpec((1, 1, L), lambda b: (b, 0, 0)),
            full((D, D)), full((D, D)), full((D, D)), full((D, D)),
            full((1, D)), full((1, D)), full((1, D)), full((1, D)),
        ],
        out_specs=pl.BlockSpec((1, L, D), lambda b: (b, 0, 0)),
        out_shape=jax.ShapeDtypeStruct((B, L, D), F32),
    )(byte_repr.astype(BF16), pb3,
      Wq.T.astype(BF16), Wk.T.astype(BF16), Wv.T.astype(BF16),
      Wo.T.astype(BF16),
      bq.reshape(1, D), bk.reshape(1, D), bv.reshape(1, D), bo.reshape(1, D))
    return out


# speedup vs baseline: 5.6270x; 1.0004x over previous
import jax
import jax.numpy as jnp
from jax.experimental import pallas as pl

B, L, D, H = 8, 2048, 768, 8
DH = D // H
ST = 256
NT = L // ST
SCALE = 1.0 / (DH ** 0.5)
F32 = jnp.float32
BF16 = jnp.bfloat16


def _dot(a, b):
    return jax.lax.dot_general(a, b, (((1,), (0,)), ((), ())),
                               preferred_element_type=F32)


def _dot_t(a, b):
    return jax.lax.dot_general(a, b, (((0,), (0,)), ((), ())),
                               preferred_element_type=F32)


def _body(x_ref, pb_ref, wq_ref, wk_ref, wv_ref, wo_ref,
          bq_ref, bk_ref, bv_ref, bo_ref, o_ref):
    x = x_ref[0]
    pb = pb_ref[0]

    bnd = (pb != 0).astype(BF16)
    seg_parts = []
    for t in range(NT):
        tril = (jax.lax.broadcasted_iota(jnp.int32, (L, ST), 0) <=
                jax.lax.broadcasted_iota(jnp.int32, (L, ST), 1) + t * ST
                ).astype(BF16)
        seg_parts.append(_dot(bnd, tril))
    seg = jnp.concatenate(seg_parts, axis=1) - bnd[:, :1].astype(F32)
    seg_i = seg.astype(jnp.int32)

    def onehot(t):
        i0 = jax.lax.broadcasted_iota(jnp.int32, (ST, L), 0) + t * ST
        return (i0 == jnp.broadcast_to(seg_i, (ST, L))).astype(BF16)

    p_tiles = [onehot(t) for t in range(NT)]

    ones_col = jnp.ones((L, 1), BF16)

    xq = _dot(x, wq_ref[...]).astype(BF16)
    q_tiles, cnt_tiles = [], []
    for t in range(NT):
        p = p_tiles[t]
        cnt = _dot(p, ones_col)
        sums = _dot(p, xq)
        q = sums / jnp.maximum(cnt, 1.0) + bq_ref[...]
        q_tiles.append(q.astype(BF16))
        cnt_tiles.append(cnt)

    qj = sum(_dot_t(p_tiles[t], q_tiles[t]) for t in range(NT))

    e_mat = (jax.lax.broadcasted_iota(jnp.int32, (D, H), 0) // DH ==
             jax.lax.broadcasted_iota(jnp.int32, (D, H), 1)).astype(F32)
    logit_tiles = []
    for t in range(NT):
        sl = slice(t * ST, (t + 1) * ST)
        k_t = _dot(x[sl], wk_ref[...]) + bk_ref[...]
        logit_tiles.append(_dot(qj[sl] * k_t, e_mat) * SCALE)
    logits = jnp.concatenate(logit_tiles, axis=0)
    m = jnp.max(logits, axis=0, keepdims=True)
    e = jnp.exp(logits - m)

    escale = jnp.zeros((L, H), F32)
    for t in range(NT):
        p = p_tiles[t]
        dinv = 1.0 / jnp.maximum(_dot(p, e.astype(BF16)), 1e-30)
        escale = escale + _dot_t(p, dinv.astype(BF16))

    w = _dot((e * escale).astype(BF16), e_mat.T.astype(BF16))
    yv_tiles = []
    for t in range(NT):
        sl = slice(t * ST, (t + 1) * ST)
        v_t = _dot(x[sl], wv_ref[...]) + bv_ref[...]
        yv_tiles.append((w[sl] * v_t).astype(BF16))
    yv = jnp.concatenate(yv_tiles, axis=0)

    for t in range(NT):
        attn = _dot(p_tiles[t], yv).astype(BF16)
        out = _dot(attn, wo_ref[...]) + bo_ref[...]
        out = jnp.where(cnt_tiles[t] > 0.0, out, 0.0)
        o_ref[0, t * ST:(t + 1) * ST, :] = out


@jax.jit
def kernel(byte_repr, patch_boundaries, Wq, Wk, Wv, bq, bk, bv, Wo, bo):
    pb3 = patch_boundaries.reshape(B, 1, L)
    full = lambda shape: pl.BlockSpec(shape, lambda b: (0,) * len(shape))
    out = pl.pallas_call(
        _body,
        grid=(B,),
        in_specs=[
            pl.BlockSpec((1, L, D), lambda b: (b, 0, 0)),
            pl.BlockSpec((1, 1, L), lambda b: (b, 0, 0)),
            full((D, D)), full((D, D)), full((D, D)), full((D, D)),
            full((1, D)), full((1, D)), full((1, D)), full((1, D)),
        ],
        out_specs=pl.BlockSpec((1, L, D), lambda b: (b, 0, 0)),
        out_shape=jax.ShapeDtypeStruct((B, L, D), F32),
    )(byte_repr.astype(BF16), pb3,
      Wq.T.astype(BF16), Wk.T.astype(BF16), Wv.T.astype(BF16),
      Wo.T.astype(BF16),
      bq.reshape(1, D), bk.reshape(1, D), bv.reshape(1, D), bo.reshape(1, D))
    return out


# speedup vs baseline: 9.1816x; 1.6317x over previous
import jax
import jax.numpy as jnp
from jax.experimental import pallas as pl

B, L, D, H = 8, 2048, 768, 8
DH = D // H
ST = 256
NT = L // ST
SCALE = 1.0 / (DH ** 0.5)
F32 = jnp.float32
BF16 = jnp.bfloat16


def _dot(a, b):
    return jax.lax.dot_general(a, b, (((1,), (0,)), ((), ())),
                               preferred_element_type=F32)


def _body(x_ref, pb_ref, wq_ref, wk_ref, wv_ref, wo_ref,
          bq_ref, bk_ref, bv_ref, bo_ref, o_ref):
    x = x_ref[0]
    pb = pb_ref[0]

    bnd = (pb != 0).astype(BF16)
    seg_parts = []
    for t in range(NT):
        tril = (jax.lax.broadcasted_iota(jnp.int32, (L, ST), 0) <=
                jax.lax.broadcasted_iota(jnp.int32, (L, ST), 1) + t * ST
                ).astype(BF16)
        seg_parts.append(_dot(bnd, tril))
    seg = jnp.concatenate(seg_parts, axis=1) - bnd[:, :1].astype(F32)
    seg_i = seg.astype(jnp.int32)
    seg_col = jnp.reshape(seg_i, (L, 1))
    nt_used = seg_i[0, L - 1] // ST + 1

    def a_tile(t):
        rows = seg_col[t * ST:(t + 1) * ST]
        return (rows == jnp.broadcast_to(seg_i, (ST, L))).astype(BF16)

    def onehot(t):
        i0 = jax.lax.broadcasted_iota(jnp.int32, (ST, L), 0) + t * ST
        return (i0 == jnp.broadcast_to(seg_i, (ST, L))).astype(BF16)

    ones_col = jnp.ones((L, 1), BF16)
    e_mat = (jax.lax.broadcasted_iota(jnp.int32, (D, H), 0) // DH ==
             jax.lax.broadcasted_iota(jnp.int32, (D, H), 1)).astype(BF16)

    xq = _dot(x, wq_ref[...]).astype(BF16)
    logit_tiles = []
    for t in range(NT):
        sl = slice(t * ST, (t + 1) * ST)
        a = a_tile(t)
        cpos = _dot(a, ones_col)
        qj = _dot(a, xq) / cpos + bq_ref[...]
        k_t = _dot(x[sl], wk_ref[...]) + bk_ref[...]
        prod = (qj * k_t).astype(BF16)
        logit_tiles.append(_dot(prod, e_mat) * SCALE)
    logits = jnp.concatenate(logit_tiles, axis=0)
    m = jnp.max(logits, axis=0, keepdims=True)
    e = jnp.exp(logits - m)
    e16 = e.astype(BF16)

    yv_tiles = []
    for t in range(NT):
        sl = slice(t * ST, (t + 1) * ST)
        a = a_tile(t)
        denpos = _dot(a, e16)
        wgt = e[sl] / jnp.maximum(denpos, 1e-30)
        wexp = _dot(wgt.astype(BF16), e_mat.T)
        v_t = _dot(x[sl], wv_ref[...]) + bv_ref[...]
        yv_tiles.append((wexp * v_t).astype(BF16))
    yv = jnp.concatenate(yv_tiles, axis=0)

    for t in range(NT):
        osl = (0, t * ST)

        @pl.when(t < nt_used)
        def _store():
            p = onehot(t)
            cnt = _dot(p, ones_col)
            attn = _dot(p, yv).astype(BF16)
            out = _dot(attn, wo_ref[...]) + bo_ref[...]
            out = jnp.where(cnt > 0.0, out, 0.0)
            o_ref[0, t * ST:(t + 1) * ST, :] = out

        @pl.when(t >= nt_used)
        def _zero():
            o_ref[0, t * ST:(t + 1) * ST, :] = jnp.zeros((ST, D), F32)


@jax.jit
def kernel(byte_repr, patch_boundaries, Wq, Wk, Wv, bq, bk, bv, Wo, bo):
    pb3 = patch_boundaries.reshape(B, 1, L)
    full = lambda shape: pl.BlockSpec(shape, lambda b: (0,) * len(shape))
    out = pl.pallas_call(
        _body,
        grid=(B,),
        in_specs=[
            pl.BlockSpec((1, L, D), lambda b: (b, 0, 0)),
            pl.BlockSpec((1, 1, L), lambda b: (b, 0, 0)),
            full((D, D)), full((D, D)), full((D, D)), full((D, D)),
            full((1, D)), full((1, D)), full((1, D)), full((1, D)),
        ],
        out_specs=pl.BlockSpec((1, L, D), lambda b: (b, 0, 0)),
        out_shape=jax.ShapeDtypeStruct((B, L, D), F32),
    )(byte_repr.astype(BF16), pb3,
      Wq.T.astype(BF16), Wk.T.astype(BF16), Wv.T.astype(BF16),
      Wo.T.astype(BF16),
      bq.reshape(1, D), bk.reshape(1, D), bv.reshape(1, D), bo.reshape(1, D))
    return out


# speedup vs baseline: 9.5910x; 1.0446x over previous
import jax
import jax.numpy as jnp
from jax.experimental import pallas as pl
from jax.experimental.pallas import tpu as pltpu

B, L, D, H = 8, 2048, 768, 8
DH = D // H
ST = 256
NT = L // ST
SCALE = 1.0 / (DH ** 0.5)
F32 = jnp.float32
BF16 = jnp.bfloat16


def _dot(a, b):
    return jax.lax.dot_general(a, b, (((1,), (0,)), ((), ())),
                               preferred_element_type=F32)


def _body(x_ref, pb_ref, wq_ref, wk_ref, wv_ref, wo_ref,
          bq_ref, bk_ref, bv_ref, bo_ref, o_ref, seg_sc):
    pid = pl.program_id(0)

    @pl.when(pid == 0)
    def _seg_all():
        bnd_all = (jnp.reshape(pb_ref[...], (B, L)) != 0).astype(BF16)
        parts = []
        for t in range(NT):
            tril = (jax.lax.broadcasted_iota(jnp.int32, (L, ST), 0) <=
                    jax.lax.broadcasted_iota(jnp.int32, (L, ST), 1) + t * ST
                    ).astype(BF16)
            parts.append(_dot(bnd_all, tril))
        seg_all = (jnp.concatenate(parts, axis=1) -
                   bnd_all[:, :1].astype(F32))
        seg_sc[...] = seg_all.astype(jnp.int32)

    x = x_ref[0]
    seg_i = seg_sc[pl.ds(pid, 1), :]
    seg_col = jnp.reshape(seg_i, (L, 1))
    maxseg = seg_i[0, L - 1]
    nt_used = maxseg // ST + 1

    def a_tile(t):
        rows = seg_col[t * ST:(t + 1) * ST]
        return (rows == jnp.broadcast_to(seg_i, (ST, L))).astype(BF16)

    def onehot(t):
        i0 = jax.lax.broadcasted_iota(jnp.int32, (ST, L), 0) + t * ST
        return (i0 == jnp.broadcast_to(seg_i, (ST, L))).astype(BF16)

    e_mat = (jax.lax.broadcasted_iota(jnp.int32, (D, H), 0) // DH ==
             jax.lax.broadcasted_iota(jnp.int32, (D, H), 1)).astype(BF16)

    xq = _dot(x, wq_ref[...]).astype(BF16)
    xaug = jnp.concatenate([xq, jnp.ones((L, 1), BF16)], axis=1)
    logit_tiles = []
    for t in range(NT):
        sl = slice(t * ST, (t + 1) * ST)
        a = a_tile(t)
        r = _dot(a, xaug)
        qj = r[:, :D] / r[:, D:] + bq_ref[...]
        k_t = _dot(x[sl], wk_ref[...]) + bk_ref[...]
        prod = (qj * k_t).astype(BF16)
        logit_tiles.append(_dot(prod, e_mat) * SCALE)
    logits = jnp.concatenate(logit_tiles, axis=0)
    m = jnp.max(logits, axis=0, keepdims=True)
    e = jnp.exp(logits - m)
    e16 = e.astype(BF16)

    yv_tiles = []
    for t in range(NT):
        sl = slice(t * ST, (t + 1) * ST)
        a = a_tile(t)
        denpos = _dot(a, e16)
        wgt = e[sl] / jnp.maximum(denpos, 1e-30)
        wexp = _dot(wgt.astype(BF16), e_mat.T)
        v_t = _dot(x[sl], wv_ref[...]) + bv_ref[...]
        yv_tiles.append((wexp * v_t).astype(BF16))
    yv = jnp.concatenate(yv_tiles, axis=0)

    row_id = jax.lax.broadcasted_iota(jnp.int32, (ST, 1), 0)
    for t in range(NT):

        @pl.when(t < nt_used)
        def _store():
            p = onehot(t)
            attn = _dot(p, yv).astype(BF16)
            out = _dot(attn, wo_ref[...]) + bo_ref[...]
            out = jnp.where(row_id + t * ST <= maxseg, out, 0.0)
            o_ref[0, t * ST:(t + 1) * ST, :] = out

        @pl.when(t >= nt_used)
        def _zero():
            o_ref[0, t * ST:(t + 1) * ST, :] = jnp.zeros((ST, D), F32)


@jax.jit
def kernel(byte_repr, patch_boundaries, Wq, Wk, Wv, bq, bk, bv, Wo, bo):
    pb3 = patch_boundaries.reshape(B, 1, L)
    full = lambda shape: pl.BlockSpec(shape, lambda b: (0,) * len(shape))
    out = pl.pallas_call(
        _body,
        grid=(B,),
        in_specs=[
            pl.BlockSpec((1, L, D), lambda b: (b, 0, 0)),
            full((B, 1, L)),
            full((D, D)), full((D, D)), full((D, D)), full((D, D)),
            full((1, D)), full((1, D)), full((1, D)), full((1, D)),
        ],
        out_specs=pl.BlockSpec((1, L, D), lambda b: (b, 0, 0)),
        out_shape=jax.ShapeDtypeStruct((B, L, D), F32),
        scratch_shapes=[pltpu.VMEM((B, L), jnp.int32)],
    )(byte_repr.astype(BF16), pb3,
      Wq.T.astype(BF16), Wk.T.astype(BF16), Wv.T.astype(BF16),
      Wo.T.astype(BF16),
      bq.reshape(1, D), bk.reshape(1, D), bv.reshape(1, D), bo.reshape(1, D))
    return out
